# baseline (speedup 1.0000x reference)
import functools

import jax
import jax.numpy as jnp
from jax import lax
from jax.experimental import pallas as pl
from jax.experimental.pallas import tpu as pltpu
from jax.experimental.pallas import tpu_sc as plsc

N = 10000
D = 128
E = 320000
HID = 128

NC = 2
NS = 16
NW = NC * NS
LANES = 16
CHUNK = 128
CH_PER_W = -(-E // (CHUNK * NW))
NCH = CH_PER_W * NW
EPAD = NCH * CHUNK - E
NBUF = 2
NPAD = 10240
ROWS_PER_S = NPAD // NS
ZCHUNKS = ROWS_PER_S // CHUNK


def _sc_aggregate(x, eidx, z128):
    mesh = plsc.VectorSubcoreMesh(core_axis_name="c", subcore_axis_name="s")

    @functools.partial(
        pl.kernel,
        out_type=(
            jax.ShapeDtypeStruct((NC, NPAD, D), jnp.float32),
            jax.ShapeDtypeStruct((NC, NS, NPAD), jnp.float32),
        ),
        mesh=mesh,
        compiler_params=pltpu.CompilerParams(needs_layout_passes=False),
        scratch_types=[
            pltpu.VMEM_SHARED((NPAD, D), jnp.float32),
            pltpu.VMEM((NBUF, 2, CHUNK), jnp.int32),
            pltpu.VMEM((NBUF, CHUNK, D), jnp.float32),
            pltpu.VMEM((NPAD,), jnp.float32),
            pltpu.SemaphoreType.DMA,
            pltpu.SemaphoreType.DMA,
            pltpu.SemaphoreType.DMA,
        ],
    )
    def k(x_hbm, eidx_hbm, z128_hbm,
          agg_out, deg_out,
          agg_sh, idx_v, rows_v, deg_v, gsem, ssem0, ssem1):
        c = lax.axis_index("c")
        s = lax.axis_index("s")
        wid = s * NC + c

        pltpu.sync_copy(z128_hbm, rows_v.at[0])
        r0 = s * ROWS_PER_S

        def zinit(i, carry):
            pltpu.sync_copy(rows_v.at[0],
                            agg_sh.at[pl.ds(r0 + i * CHUNK, CHUNK)])
            return carry

        lax.fori_loop(0, ZCHUNKS, zinit, 0)

        zeros16 = jnp.zeros((LANES,), jnp.float32)

        def zdeg(i, carry):
            deg_v[pl.ds(i * LANES, LANES)] = zeros16
            return carry

        lax.fori_loop(0, NPAD // LANES, zdeg, 0)
        plsc.subcore_barrier()

        cbase = wid * CH_PER_W
        ones16 = jnp.full((LANES,), 1.0, jnp.float32)

        ssems = (ssem0, ssem1)

        def chunk_step(j, b, drain):
            if drain:
                pltpu.make_async_copy(rows_v.at[b],
                                      agg_sh.at[idx_v.at[b, 0]],
                                      ssems[b]).wait()
            pltpu.sync_copy(eidx_hbm.at[cbase + j], idx_v.at[b])
            pltpu.async_copy(x_hbm.at[idx_v.at[b, 1]], rows_v.at[b],
                             gsem).wait()
            pltpu.async_copy(rows_v.at[b], agg_sh.at[idx_v.at[b, 0]],
                             ssems[b], add=True)
            for kk in range(CHUNK // LANES):
                idx = idx_v[b, 0, pl.ds(kk * LANES, LANES)]
                plsc.addupdate_scatter(deg_v, [idx], ones16)

        chunk_step(0, 0, drain=False)
        chunk_step(1, 1, drain=False)

        def body(t, carry):
            j = NBUF + t * NBUF
            chunk_step(j, 0, drain=True)
            chunk_step(j + 1, 1, drain=True)
            return carry

        npairs = (CH_PER_W - NBUF) // NBUF
        lax.fori_loop(0, npairs, body, 0)
        chunk_step(CH_PER_W - 1, 0, drain=True)
        pltpu.make_async_copy(rows_v.at[0], agg_sh.at[idx_v.at[0, 0]],
                              ssem0).wait()
        pltpu.make_async_copy(rows_v.at[1], agg_sh.at[idx_v.at[1, 0]],
                              ssem1).wait()
        plsc.subcore_barrier()

        def wout(i, carry):
            rr = r0 + i * CHUNK
            pltpu.sync_copy(agg_sh.at[pl.ds(rr, CHUNK)], rows_v.at[0])
            pltpu.sync_copy(rows_v.at[0], agg_out.at[c, pl.ds(rr, CHUNK)])
            return carry

        lax.fori_loop(0, ZCHUNKS, wout, 0)
        pltpu.sync_copy(deg_v, deg_out.at[c, s])

    return k(x, eidx, z128)


TC_ROWS = 1000


def _tc_mlp_body(x_ref, agg_ref, deg_ref, w1a_ref, w1b_ref, w2_ref, w3_ref,
                 w4_ref, b1_ref, b2_ref, b3_ref, b4_ref, out_ref):
    deg = jnp.sum(deg_ref[...], axis=1, keepdims=True)
    agg = (agg_ref[0] + agg_ref[1]) * (1.0 / jnp.maximum(deg, 1.0))
    f32 = jnp.float32
    h = jnp.maximum(
        jnp.dot(x_ref[...], w1a_ref[...], preferred_element_type=f32)
        + jnp.dot(agg, w1b_ref[...], preferred_element_type=f32)
        + b1_ref[...], 0.0)
    h = jnp.maximum(
        jnp.dot(h, w2_ref[...], preferred_element_type=f32) + b2_ref[...], 0.0)
    h = jnp.maximum(
        jnp.dot(h, w3_ref[...], preferred_element_type=f32) + b3_ref[...], 0.0)
    out_ref[...] = (
        jnp.dot(h, w4_ref[...], preferred_element_type=f32) + b4_ref[...])


def _tc_mlp(x, agg_p, deg_t, w1a, w1b, w2, w3, w4, b1, b2, b3, b4):
    grid = (N // TC_ROWS,)
    full = lambda shape: pl.BlockSpec(shape, lambda i: (0,) * len(shape))
    return pl.pallas_call(
        _tc_mlp_body,
        grid=grid,
        in_specs=[
            pl.BlockSpec((TC_ROWS, D), lambda i: (i, 0)),
            pl.BlockSpec((NC, TC_ROWS, D), lambda i: (0, i, 0)),
            pl.BlockSpec((TC_ROWS, NW), lambda i: (i, 0)),
            full((D, HID)), full((D, HID)), full((HID, HID)),
            full((HID, HID)), full((HID, D)),
            full((1, HID)), full((1, HID)), full((1, HID)), full((1, D)),
        ],
        out_specs=pl.BlockSpec((TC_ROWS, D), lambda i: (i, 0)),
        out_shape=jax.ShapeDtypeStruct((N, D), jnp.float32),
    )(x, agg_p, deg_t, w1a, w1b, w2, w3, w4, b1, b2, b3, b4)


def kernel(x, edge_index, W1, b1, W2, b2, W3, b3, W4, b4):
    pad = jnp.stack([jnp.full((EPAD,), N, jnp.int32),
                     jnp.zeros((EPAD,), jnp.int32)])
    eidx = jnp.concatenate([edge_index.astype(jnp.int32), pad], axis=1)
    eidx = jnp.transpose(eidx.reshape(2, NCH, CHUNK), (1, 0, 2))
    z128 = jnp.zeros((CHUNK, D), jnp.float32)
    agg_p, deg_p = _sc_aggregate(x, eidx, z128)
    deg_t = jnp.transpose(deg_p.reshape(NC * NS, NPAD))
    w1a = W1[:, :D].T
    w1b = W1[:, D:].T
    return _tc_mlp(x, agg_p, deg_t, w1a, w1b, W2.T, W3.T, W4.T,
                   b1.reshape(1, HID), b2.reshape(1, HID),
                   b3.reshape(1, HID), b4.reshape(1, D))


# speedup vs baseline: 1.7077x; 1.7077x over previous
import functools

import jax
import jax.numpy as jnp
from jax import lax
from jax.experimental import pallas as pl
from jax.experimental.pallas import tpu as pltpu
from jax.experimental.pallas import tpu_sc as plsc

N = 10000
D = 128
E = 320000
HID = 128

NC = 2
NS = 16
NW = NC * NS
LANES = 16
CHUNK = 80
NCH = E // CHUNK
CH_PER_W = NCH // NW
NBUF = 3
NPAD = 10240
ROWS_PER_S = NPAD // NS
ZCHUNKS = ROWS_PER_S // CHUNK


def _sc_aggregate(x, eidx, z128):
    mesh = plsc.VectorSubcoreMesh(core_axis_name="c", subcore_axis_name="s")

    @functools.partial(
        pl.kernel,
        out_type=(
            jax.ShapeDtypeStruct((NC, NPAD, D), jnp.float32),
            jax.ShapeDtypeStruct((NC, NS, NPAD), jnp.float32),
        ),
        mesh=mesh,
        compiler_params=pltpu.CompilerParams(needs_layout_passes=False),
        scratch_types=[
            pltpu.VMEM_SHARED((NPAD, D), jnp.float32),
            pltpu.VMEM((NBUF, 2, CHUNK), jnp.int32),
            pltpu.VMEM((NBUF, CHUNK, D), jnp.float32),
            pltpu.VMEM((NPAD,), jnp.float32),
            pltpu.SemaphoreType.DMA,
            pltpu.SemaphoreType.DMA,
            pltpu.SemaphoreType.DMA,
            pltpu.SemaphoreType.DMA,
            pltpu.SemaphoreType.DMA,
            pltpu.SemaphoreType.DMA,
            pltpu.SemaphoreType.DMA,
        ],
    )
    def k(x_hbm, eidx_hbm, z128_hbm,
          agg_out, deg_out,
          agg_sh, idx_v, rows_v, deg_v, gsem, ssem0, ssem1, ssem2,
          isem0, isem1, isem2):
        c = lax.axis_index("c")
        s = lax.axis_index("s")
        wid = s * NC + c

        pltpu.sync_copy(z128_hbm, rows_v.at[0])
        r0 = s * ROWS_PER_S

        def zinit(i, carry):
            pltpu.sync_copy(rows_v.at[0],
                            agg_sh.at[pl.ds(r0 + i * CHUNK, CHUNK)])
            return carry

        lax.fori_loop(0, ZCHUNKS, zinit, 0)

        zeros16 = jnp.zeros((LANES,), jnp.float32)

        def zdeg(i, carry):
            deg_v[pl.ds(i * LANES, LANES)] = zeros16
            return carry

        lax.fori_loop(0, NPAD // LANES, zdeg, 0)
        plsc.subcore_barrier()

        cbase = wid * CH_PER_W
        ones16 = jnp.full((LANES,), 1.0, jnp.float32)

        ssems = (ssem0, ssem1, ssem2)
        isems = (isem0, isem1, isem2)

        def chunk_step(j, b, drain):
            bn = (b + 1) % NBUF
            pltpu.make_async_copy(eidx_hbm.at[cbase], idx_v.at[b],
                                  isems[b]).wait()
            if drain:
                pltpu.make_async_copy(rows_v.at[bn],
                                      agg_sh.at[idx_v.at[bn, 0]],
                                      ssems[bn]).wait()
            jn = jnp.minimum(cbase + j + 1, NCH - 1)
            pltpu.async_copy(eidx_hbm.at[jn], idx_v.at[bn], isems[bn])
            pltpu.async_copy(x_hbm.at[idx_v.at[b, 1]], rows_v.at[b],
                             gsem).wait()
            pltpu.async_copy(rows_v.at[b], agg_sh.at[idx_v.at[b, 0]],
                             ssems[b], add=True)
            for kk in range(CHUNK // LANES):
                idx = idx_v[b, 0, pl.ds(kk * LANES, LANES)]
                plsc.addupdate_scatter(deg_v, [idx], ones16)

        pltpu.async_copy(eidx_hbm.at[cbase], idx_v.at[0], isem0)
        chunk_step(0, 0, drain=False)
        chunk_step(1, 1, drain=False)

        def body(t, carry):
            j = 2 + t * NBUF
            chunk_step(j, 2, drain=True)
            chunk_step(j + 1, 0, drain=True)
            chunk_step(j + 2, 1, drain=True)
            return carry

        ntriples = (CH_PER_W - 2) // NBUF
        lax.fori_loop(0, ntriples, body, 0)
        pltpu.make_async_copy(rows_v.at[0], agg_sh.at[idx_v.at[0, 0]],
                              ssem0).wait()
        pltpu.make_async_copy(rows_v.at[1], agg_sh.at[idx_v.at[1, 0]],
                              ssem1).wait()
        pltpu.make_async_copy(eidx_hbm.at[cbase], idx_v.at[2], isem2).wait()
        plsc.subcore_barrier()

        def wout(i, carry):
            rr = r0 + i * CHUNK
            pltpu.sync_copy(agg_sh.at[pl.ds(rr, CHUNK)], rows_v.at[0])
            pltpu.sync_copy(rows_v.at[0], agg_out.at[c, pl.ds(rr, CHUNK)])
            return carry

        lax.fori_loop(0, ZCHUNKS, wout, 0)
        pltpu.sync_copy(deg_v, deg_out.at[c, s])

    return k(x, eidx, z128)


TC_ROWS = 1000


def _tc_mlp_body(x_ref, agg_ref, deg_ref, w1a_ref, w1b_ref, w2_ref, w3_ref,
                 w4_ref, b1_ref, b2_ref, b3_ref, b4_ref, out_ref):
    deg = jnp.sum(deg_ref[...], axis=1, keepdims=True)
    agg = (agg_ref[0] + agg_ref[1]) * (1.0 / jnp.maximum(deg, 1.0))
    f32 = jnp.float32
    h = jnp.maximum(
        jnp.dot(x_ref[...], w1a_ref[...], preferred_element_type=f32)
        + jnp.dot(agg, w1b_ref[...], preferred_element_type=f32)
        + b1_ref[...], 0.0)
    h = jnp.maximum(
        jnp.dot(h, w2_ref[...], preferred_element_type=f32) + b2_ref[...], 0.0)
    h = jnp.maximum(
        jnp.dot(h, w3_ref[...], preferred_element_type=f32) + b3_ref[...], 0.0)
    out_ref[...] = (
        jnp.dot(h, w4_ref[...], preferred_element_type=f32) + b4_ref[...])


def _tc_mlp(x, agg_p, deg_t, w1a, w1b, w2, w3, w4, b1, b2, b3, b4):
    grid = (N // TC_ROWS,)
    full = lambda shape: pl.BlockSpec(shape, lambda i: (0,) * len(shape))
    return pl.pallas_call(
        _tc_mlp_body,
        grid=grid,
        in_specs=[
            pl.BlockSpec((TC_ROWS, D), lambda i: (i, 0)),
            pl.BlockSpec((NC, TC_ROWS, D), lambda i: (0, i, 0)),
            pl.BlockSpec((TC_ROWS, NW), lambda i: (i, 0)),
            full((D, HID)), full((D, HID)), full((HID, HID)),
            full((HID, HID)), full((HID, D)),
            full((1, HID)), full((1, HID)), full((1, HID)), full((1, D)),
        ],
        out_specs=pl.BlockSpec((TC_ROWS, D), lambda i: (i, 0)),
        out_shape=jax.ShapeDtypeStruct((N, D), jnp.float32),
    )(x, agg_p, deg_t, w1a, w1b, w2, w3, w4, b1, b2, b3, b4)


def kernel(x, edge_index, W1, b1, W2, b2, W3, b3, W4, b4):
    eidx = jnp.transpose(
        edge_index.astype(jnp.int32).reshape(2, NCH, CHUNK), (1, 0, 2))
    z128 = jnp.zeros((CHUNK, D), jnp.float32)
    agg_p, deg_p = _sc_aggregate(x, eidx, z128)
    deg_t = jnp.transpose(deg_p.reshape(NC * NS, NPAD))
    w1a = W1[:, :D].T
    w1b = W1[:, D:].T
    return _tc_mlp(x, agg_p, deg_t, w1a, w1b, W2.T, W3.T, W4.T,
                   b1.reshape(1, HID), b2.reshape(1, HID),
                   b3.reshape(1, HID), b4.reshape(1, D))


# speedup vs baseline: 1.7320x; 1.0142x over previous
import functools

import jax
import jax.numpy as jnp
from jax import lax
from jax.experimental import pallas as pl
from jax.experimental.pallas import tpu as pltpu
from jax.experimental.pallas import tpu_sc as plsc

N = 10000
D = 128
E = 320000
HID = 128

NC = 2
NS = 16
NW = NC * NS
LANES = 16
CHUNK = 80
NCH = E // CHUNK
CH_PER_W = NCH // NW
NBUF = 3
NPAD = 10240
ROWS_PER_S = NPAD // NS
ZCHUNKS = ROWS_PER_S // CHUNK


def _sc_aggregate(x, eidx, z128):
    mesh = plsc.VectorSubcoreMesh(core_axis_name="c", subcore_axis_name="s")

    @functools.partial(
        pl.kernel,
        out_type=(
            jax.ShapeDtypeStruct((NC, NPAD, D), jnp.float32),
            jax.ShapeDtypeStruct((NC, NS, NPAD), jnp.float32),
        ),
        mesh=mesh,
        compiler_params=pltpu.CompilerParams(needs_layout_passes=False),
        scratch_types=[
            pltpu.VMEM_SHARED((NPAD, D), jnp.float32),
            pltpu.VMEM((4, 2, CHUNK), jnp.int32),
            pltpu.VMEM((3, CHUNK, D), jnp.float32),
            pltpu.VMEM((NPAD,), jnp.float32),
            pltpu.SemaphoreType.DMA,
            pltpu.SemaphoreType.DMA,
            pltpu.SemaphoreType.DMA,
            pltpu.SemaphoreType.DMA,
            pltpu.SemaphoreType.DMA,
            pltpu.SemaphoreType.DMA,
            pltpu.SemaphoreType.DMA,
            pltpu.SemaphoreType.DMA,
        ],
    )
    def k(x_hbm, eidx_hbm, z128_hbm,
          agg_out, deg_out,
          agg_sh, idx_v, rows_v, deg_v, gsem, ssem0, ssem1, ssem2,
          isem0, isem1, isem2, isem3):
        c = lax.axis_index("c")
        s = lax.axis_index("s")
        wid = s * NC + c

        pltpu.sync_copy(z128_hbm, rows_v.at[0])
        r0 = s * ROWS_PER_S

        def zinit(i, carry):
            pltpu.sync_copy(rows_v.at[0],
                            agg_sh.at[pl.ds(r0 + i * CHUNK, CHUNK)])
            return carry

        lax.fori_loop(0, ZCHUNKS, zinit, 0)

        zeros16 = jnp.zeros((LANES,), jnp.float32)

        def zdeg(i, carry):
            deg_v[pl.ds(i * LANES, LANES)] = zeros16
            return carry

        lax.fori_loop(0, NPAD // LANES, zdeg, 0)
        plsc.subcore_barrier()

        cbase = wid * CH_PER_W
        ones16 = jnp.full((LANES,), 1.0, jnp.float32)

        ssems = (ssem0, ssem1, ssem2)
        isems = (isem0, isem1, isem2, isem3)

        def wait_idx(q):
            pltpu.make_async_copy(eidx_hbm.at[cbase], idx_v.at[q],
                                  isems[q]).wait()

        def wait_gather(r):
            pltpu.make_async_copy(x_hbm.at[idx_v.at[0, 1]], rows_v.at[r],
                                  gsem).wait()

        def drain_scatter(r):
            pltpu.make_async_copy(rows_v.at[r], agg_sh.at[idx_v.at[0, 0]],
                                  ssems[r]).wait()

        def start_gather(j, r, q):
            pltpu.async_copy(x_hbm.at[idx_v.at[q, 1]], rows_v.at[r], gsem)

        def start_scatter(r, q):
            pltpu.async_copy(rows_v.at[r], agg_sh.at[idx_v.at[q, 0]],
                             ssems[r], add=True)

        def prefetch_idx(j, q):
            pltpu.async_copy(eidx_hbm.at[cbase + j], idx_v.at[q], isems[q])

        def hist(q):
            for kk in range(CHUNK // LANES):
                idx = idx_v[q, 0, pl.ds(kk * LANES, LANES)]
                plsc.addupdate_scatter(deg_v, [idx], ones16)

        def step(j, js, drain, pref, gnext):
            r, q = js % 3, js % 4
            wait_gather(r)
            start_scatter(r, q)
            if gnext:
                wait_idx((js + 1) % 4)
            if drain:
                drain_scatter((js + 1) % 3)
            if gnext:
                start_gather(j + 1, (js + 1) % 3, (js + 1) % 4)
            if pref:
                prefetch_idx(j + 2, (js + 2) % 4)
            hist(q)

        prefetch_idx(0, 0)
        prefetch_idx(1, 1)
        wait_idx(0)
        start_gather(0, 0, 0)
        step(0, 0, drain=False, pref=True, gnext=True)
        step(1, 1, drain=False, pref=True, gnext=True)

        def body(t, carry):
            j0 = 2 + t * 12
            for u in range(12):
                step(j0 + u, 2 + u, drain=True, pref=True, gnext=True)
            return carry

        lax.fori_loop(0, (CH_PER_W - 5) // 12, body, 0)
        step(122, 122, drain=True, pref=True, gnext=True)
        step(123, 123, drain=True, pref=False, gnext=True)
        step(124, 124, drain=True, pref=False, gnext=False)
        drain_scatter(123 % 3)
        drain_scatter(124 % 3)
        plsc.subcore_barrier()

        def wout(i, carry):
            rr = r0 + i * CHUNK
            pltpu.sync_copy(agg_sh.at[pl.ds(rr, CHUNK)], rows_v.at[0])
            pltpu.sync_copy(rows_v.at[0], agg_out.at[c, pl.ds(rr, CHUNK)])
            return carry

        lax.fori_loop(0, ZCHUNKS, wout, 0)
        pltpu.sync_copy(deg_v, deg_out.at[c, s])

    return k(x, eidx, z128)


TC_ROWS = 1000


def _tc_mlp_body(x_ref, agg_ref, deg_ref, w1a_ref, w1b_ref, w2_ref, w3_ref,
                 w4_ref, b1_ref, b2_ref, b3_ref, b4_ref, out_ref):
    deg = jnp.sum(deg_ref[...], axis=1, keepdims=True)
    agg = (agg_ref[0] + agg_ref[1]) * (1.0 / jnp.maximum(deg, 1.0))
    f32 = jnp.float32
    h = jnp.maximum(
        jnp.dot(x_ref[...], w1a_ref[...], preferred_element_type=f32)
        + jnp.dot(agg, w1b_ref[...], preferred_element_type=f32)
        + b1_ref[...], 0.0)
    h = jnp.maximum(
        jnp.dot(h, w2_ref[...], preferred_element_type=f32) + b2_ref[...], 0.0)
    h = jnp.maximum(
        jnp.dot(h, w3_ref[...], preferred_element_type=f32) + b3_ref[...], 0.0)
    out_ref[...] = (
        jnp.dot(h, w4_ref[...], preferred_element_type=f32) + b4_ref[...])


def _tc_mlp(x, agg_p, deg_t, w1a, w1b, w2, w3, w4, b1, b2, b3, b4):
    grid = (N // TC_ROWS,)
    full = lambda shape: pl.BlockSpec(shape, lambda i: (0,) * len(shape))
    return pl.pallas_call(
        _tc_mlp_body,
        grid=grid,
        in_specs=[
            pl.BlockSpec((TC_ROWS, D), lambda i: (i, 0)),
            pl.BlockSpec((NC, TC_ROWS, D), lambda i: (0, i, 0)),
            pl.BlockSpec((TC_ROWS, NW), lambda i: (i, 0)),
            full((D, HID)), full((D, HID)), full((HID, HID)),
            full((HID, HID)), full((HID, D)),
            full((1, HID)), full((1, HID)), full((1, HID)), full((1, D)),
        ],
        out_specs=pl.BlockSpec((TC_ROWS, D), lambda i: (i, 0)),
        out_shape=jax.ShapeDtypeStruct((N, D), jnp.float32),
    )(x, agg_p, deg_t, w1a, w1b, w2, w3, w4, b1, b2, b3, b4)


def kernel(x, edge_index, W1, b1, W2, b2, W3, b3, W4, b4):
    eidx = jnp.transpose(
        edge_index.astype(jnp.int32).reshape(2, NCH, CHUNK), (1, 0, 2))
    z128 = jnp.zeros((CHUNK, D), jnp.float32)
    agg_p, deg_p = _sc_aggregate(x, eidx, z128)
    deg_t = jnp.transpose(deg_p.reshape(NC * NS, NPAD))
    w1a = W1[:, :D].T
    w1b = W1[:, D:].T
    return _tc_mlp(x, agg_p, deg_t, w1a, w1b, W2.T, W3.T, W4.T,
                   b1.reshape(1, HID), b2.reshape(1, HID),
                   b3.reshape(1, HID), b4.reshape(1, D))


# speedup vs baseline: 1.7761x; 1.0254x over previous
import functools

import jax
import jax.numpy as jnp
from jax import lax
from jax.experimental import pallas as pl
from jax.experimental.pallas import tpu as pltpu
from jax.experimental.pallas import tpu_sc as plsc

N = 10000
D = 128
E = 320000
HID = 128

NC = 2
NS = 16
NW = NC * NS
LANES = 16
CHUNK = 80
NCH = E // CHUNK
CH_PER_W = NCH // NW
NBUF = 3
NPAD = 10240
ROWS_PER_S = NPAD // NS
ZCHUNKS = ROWS_PER_S // CHUNK


def _sc_aggregate(x, eidx, z128):
    mesh = plsc.VectorSubcoreMesh(core_axis_name="c", subcore_axis_name="s")

    @functools.partial(
        pl.kernel,
        out_type=(
            jax.ShapeDtypeStruct((NC, NPAD, D), jnp.float32),
            jax.ShapeDtypeStruct((NC, NS, NPAD), jnp.float32),
        ),
        mesh=mesh,
        compiler_params=pltpu.CompilerParams(needs_layout_passes=False),
        scratch_types=[
            pltpu.VMEM_SHARED((NPAD, D), jnp.float32),
            pltpu.VMEM((4, 2, CHUNK), jnp.int32),
            pltpu.VMEM((3, CHUNK, D), jnp.float32),
            pltpu.VMEM((NPAD,), jnp.float32),
            pltpu.SemaphoreType.DMA,
            pltpu.SemaphoreType.DMA,
            pltpu.SemaphoreType.DMA,
            pltpu.SemaphoreType.DMA,
            pltpu.SemaphoreType.DMA,
            pltpu.SemaphoreType.DMA,
            pltpu.SemaphoreType.DMA,
            pltpu.SemaphoreType.DMA,
        ],
    )
    def k(x_hbm, eidx_hbm, z128_hbm,
          agg_out, deg_out,
          agg_sh, idx_v, rows_v, deg_v, gsem, ssem0, ssem1, ssem2,
          isem0, isem1, isem2, isem3):
        c = lax.axis_index("c")
        s = lax.axis_index("s")
        wid = s * NC + c

        pltpu.sync_copy(z128_hbm, rows_v.at[0])
        r0 = s * ROWS_PER_S
        for i in range(ZCHUNKS):
            pltpu.async_copy(rows_v.at[0],
                             agg_sh.at[pl.ds(r0 + i * CHUNK, CHUNK)], isem0)

        zeros16 = jnp.zeros((LANES,), jnp.float32)

        def zdeg(i, carry):
            deg_v[pl.ds(i * LANES, LANES)] = zeros16
            return carry

        lax.fori_loop(0, NPAD // LANES, zdeg, 0)
        for i in range(ZCHUNKS):
            pltpu.make_async_copy(rows_v.at[0],
                                  agg_sh.at[pl.ds(r0, CHUNK)], isem0).wait()

        cbase = wid * CH_PER_W
        ones16 = jnp.full((LANES,), 1.0, jnp.float32)

        ssems = (ssem0, ssem1, ssem2)
        isems = (isem0, isem1, isem2, isem3)

        def wait_idx(q):
            pltpu.make_async_copy(eidx_hbm.at[cbase], idx_v.at[q],
                                  isems[q]).wait()

        def wait_gather(r):
            pltpu.make_async_copy(x_hbm.at[idx_v.at[0, 1]], rows_v.at[r],
                                  gsem).wait()

        def drain_scatter(r):
            pltpu.make_async_copy(rows_v.at[r], agg_sh.at[idx_v.at[0, 0]],
                                  ssems[r]).wait()

        def start_gather(j, r, q):
            pltpu.async_copy(x_hbm.at[idx_v.at[q, 1]], rows_v.at[r], gsem)

        def start_scatter(r, q):
            pltpu.async_copy(rows_v.at[r], agg_sh.at[idx_v.at[q, 0]],
                             ssems[r], add=True)

        def prefetch_idx(j, q):
            pltpu.async_copy(eidx_hbm.at[cbase + j], idx_v.at[q], isems[q])

        def hist(q):
            for kk in range(CHUNK // LANES):
                idx = idx_v[q, 0, pl.ds(kk * LANES, LANES)]
                plsc.addupdate_scatter(deg_v, [idx], ones16)

        def step(j, js, drain, pref, gnext):
            r, q = js % 3, js % 4
            wait_gather(r)
            start_scatter(r, q)
            if gnext:
                wait_idx((js + 1) % 4)
            if drain:
                drain_scatter((js + 1) % 3)
            if gnext:
                start_gather(j + 1, (js + 1) % 3, (js + 1) % 4)
            if pref:
                prefetch_idx(j + 2, (js + 2) % 4)
            hist(q)

        prefetch_idx(0, 0)
        prefetch_idx(1, 1)
        wait_idx(0)
        start_gather(0, 0, 0)
        plsc.subcore_barrier()
        step(0, 0, drain=False, pref=True, gnext=True)
        step(1, 1, drain=False, pref=True, gnext=True)

        def body(t, carry):
            j0 = 2 + t * 12
            for u in range(12):
                step(j0 + u, 2 + u, drain=True, pref=True, gnext=True)
            return carry

        lax.fori_loop(0, (CH_PER_W - 5) // 12, body, 0)
        step(122, 122, drain=True, pref=True, gnext=True)
        step(123, 123, drain=True, pref=False, gnext=True)
        step(124, 124, drain=True, pref=False, gnext=False)
        drain_scatter(123 % 3)
        drain_scatter(124 % 3)
        plsc.subcore_barrier()

        pltpu.async_copy(deg_v, deg_out.at[c, s], isem1)
        for i in range(ZCHUNKS):
            b = i % 3
            rr = r0 + i * CHUNK
            if i >= 3:
                pltpu.make_async_copy(rows_v.at[b],
                                      agg_out.at[c, pl.ds(rr, CHUNK)],
                                      ssems[b]).wait()
            pltpu.sync_copy(agg_sh.at[pl.ds(rr, CHUNK)], rows_v.at[b])
            pltpu.async_copy(rows_v.at[b], agg_out.at[c, pl.ds(rr, CHUNK)],
                             ssems[b])
        for b in range(3):
            pltpu.make_async_copy(rows_v.at[b],
                                  agg_out.at[c, pl.ds(r0, CHUNK)],
                                  ssems[b]).wait()
        pltpu.make_async_copy(deg_v, deg_out.at[c, s], isem1).wait()

    return k(x, eidx, z128)


TC_ROWS = 1000


def _tc_mlp_body(x_ref, agg_ref, deg_ref, w1a_ref, w1b_ref, w2_ref, w3_ref,
                 w4_ref, b1_ref, b2_ref, b3_ref, b4_ref, out_ref):
    deg = jnp.sum(deg_ref[...], axis=1, keepdims=True)
    agg = (agg_ref[0] + agg_ref[1]) * (1.0 / jnp.maximum(deg, 1.0))
    f32 = jnp.float32
    h = jnp.maximum(
        jnp.dot(x_ref[...], w1a_ref[...], preferred_element_type=f32)
        + jnp.dot(agg, w1b_ref[...], preferred_element_type=f32)
        + b1_ref[...], 0.0)
    h = jnp.maximum(
        jnp.dot(h, w2_ref[...], preferred_element_type=f32) + b2_ref[...], 0.0)
    h = jnp.maximum(
        jnp.dot(h, w3_ref[...], preferred_element_type=f32) + b3_ref[...], 0.0)
    out_ref[...] = (
        jnp.dot(h, w4_ref[...], preferred_element_type=f32) + b4_ref[...])


def _tc_mlp(x, agg_p, deg_t, w1a, w1b, w2, w3, w4, b1, b2, b3, b4):
    grid = (N // TC_ROWS,)
    full = lambda shape: pl.BlockSpec(shape, lambda i: (0,) * len(shape))
    return pl.pallas_call(
        _tc_mlp_body,
        grid=grid,
        in_specs=[
            pl.BlockSpec((TC_ROWS, D), lambda i: (i, 0)),
            pl.BlockSpec((NC, TC_ROWS, D), lambda i: (0, i, 0)),
            pl.BlockSpec((TC_ROWS, NW), lambda i: (i, 0)),
            full((D, HID)), full((D, HID)), full((HID, HID)),
            full((HID, HID)), full((HID, D)),
            full((1, HID)), full((1, HID)), full((1, HID)), full((1, D)),
        ],
        out_specs=pl.BlockSpec((TC_ROWS, D), lambda i: (i, 0)),
        out_shape=jax.ShapeDtypeStruct((N, D), jnp.float32),
    )(x, agg_p, deg_t, w1a, w1b, w2, w3, w4, b1, b2, b3, b4)


def kernel(x, edge_index, W1, b1, W2, b2, W3, b3, W4, b4):
    eidx = jnp.transpose(
        edge_index.astype(jnp.int32).reshape(2, NCH, CHUNK), (1, 0, 2))
    z128 = jnp.zeros((CHUNK, D), jnp.float32)
    agg_p, deg_p = _sc_aggregate(x, eidx, z128)
    deg_t = jnp.transpose(deg_p.reshape(NC * NS, NPAD))
    w1a = W1[:, :D].T
    w1b = W1[:, D:].T
    return _tc_mlp(x, agg_p, deg_t, w1a, w1b, W2.T, W3.T, W4.T,
                   b1.reshape(1, HID), b2.reshape(1, HID),
                   b3.reshape(1, HID), b4.reshape(1, D))
